# baseline (speedup 1.0000x reference)
import dataclasses
import functools

import jax
import jax.numpy as jnp
from jax import lax
from jax.experimental import pallas as pl
from jax.experimental.pallas import tpu as pltpu
from jax.experimental.pallas import tpu_sc as plsc

N = 10000
E = 320000
D = 128

NUM_CORES = 2
NUM_SUBCORES = 16
NUM_TILES = NUM_CORES * NUM_SUBCORES
CHUNK = 80
NCH = 125
EPT = CHUNK * NCH
CNT_CHUNK = 128
CNT_NCH = 80
E_PAD = NUM_TILES * CNT_NCH * CNT_CHUNK
N_PAD = 10240
ROWS_PER_TILE = N_PAD // NUM_SUBCORES


def _sc_agg_body(x_hbm, src_hbm, dst_hbm, agg_out,
                 isrc0, isrc1, isrc2, idst0, idst1, idst2,
                 rows0, rows1, rows2, sem_a, sem_b, sem_c, agg_sh):
    cid = lax.axis_index("c")
    sid = lax.axis_index("s")
    wid = cid * NUM_SUBCORES + sid
    ebase = wid * EPT

    zero16 = jnp.zeros((16,), jnp.float32)

    @pl.loop(0, CHUNK)
    def _(r):
        @pl.loop(0, D // 16)
        def _(c):
            rows0[r, pl.ds(c * 16, 16)] = zero16

    @pl.loop(0, ROWS_PER_TILE // CHUNK)
    def _(k):
        pltpu.sync_copy(rows0, agg_sh.at[pl.ds(sid * ROWS_PER_TILE + k * CHUNK, CHUNK)])

    plsc.subcore_barrier()

    def load(hbm, j, buf):
        pltpu.sync_copy(hbm.at[pl.ds(ebase + j * CHUNK, CHUNK)], buf)

    def load_idx(j, sbuf, dbuf):
        jc = jnp.minimum(j, NCH - 1)
        load(src_hbm, jc, sbuf)
        load(dst_hbm, jc, dbuf)

    def gather(idx_buf, dst_buf, sem):
        return pltpu.async_copy(x_hbm.at[idx_buf], dst_buf, sem)

    def scatter(buf, idx_buf):
        pltpu.sync_copy(buf, agg_sh.at[idx_buf], add=True)

    load(src_hbm, 0, isrc0)
    load(dst_hbm, 0, idst0)
    pltpu.sync_copy(x_hbm.at[isrc0], rows0)
    load_idx(1, isrc1, idst1)
    load_idx(2, isrc2, idst2)

    @pl.loop(0, NCH // 3)
    def _(jj):
        j = jj * 3
        cp1 = gather(isrc1, rows1, sem_b)
        cp2 = gather(isrc2, rows2, sem_c)
        scatter(rows0, idst0)
        load_idx(j + 3, isrc0, idst0)
        cp1.wait()
        cp3 = gather(isrc0, rows0, sem_a)
        scatter(rows1, idst1)
        load_idx(j + 4, isrc1, idst1)
        cp2.wait()
        scatter(rows2, idst2)
        load_idx(j + 5, isrc2, idst2)
        cp3.wait()

    cp1 = gather(isrc1, rows1, sem_b)
    scatter(rows0, idst0)
    cp1.wait()
    scatter(rows1, idst1)

    plsc.subcore_barrier()

    pltpu.sync_copy(agg_sh.at[pl.ds(sid * ROWS_PER_TILE, ROWS_PER_TILE)],
                    agg_out.at[cid, pl.ds(sid * ROWS_PER_TILE, ROWS_PER_TILE)])


def _sc_count_body(dst_hbm, cnt_out, idx_dst, cnt_local):
    cid = lax.axis_index("c")
    sid = lax.axis_index("s")
    wid = cid * NUM_SUBCORES + sid

    zero16 = jnp.zeros((16,), jnp.float32)
    zeros16i = jnp.zeros((16,), jnp.int32)
    ones16 = jnp.ones((16,), jnp.float32)

    @pl.loop(0, N_PAD // 16)
    def _(r):
        cnt_local[0, pl.ds(r * 16, 16)] = zero16

    pltpu.sync_copy(dst_hbm.at[wid], idx_dst)

    @pl.loop(0, CNT_NCH)
    def _(j):
        @pl.loop(0, CNT_CHUNK // 16)
        def _(t):
            idx16 = idx_dst[j, pl.ds(t * 16, 16)]
            plsc.addupdate_scatter(cnt_local, [zeros16i, idx16], ones16)

    pltpu.sync_copy(cnt_local, cnt_out.at[wid])


_SC_MESH = plsc.VectorSubcoreMesh(core_axis_name="c", subcore_axis_name="s")

_SC_CP = pltpu.CompilerParams()
if "needs_layout_passes" in pltpu.CompilerParams.__dataclass_fields__:
    _SC_CP = dataclasses.replace(_SC_CP, needs_layout_passes=False)

_sc_agg = pl.kernel(
    _sc_agg_body,
    out_type=jax.ShapeDtypeStruct((NUM_CORES, N_PAD, D), jnp.float32),
    mesh=_SC_MESH,
    scratch_types=[
        pltpu.VMEM((CHUNK,), jnp.int32),
        pltpu.VMEM((CHUNK,), jnp.int32),
        pltpu.VMEM((CHUNK,), jnp.int32),
        pltpu.VMEM((CHUNK,), jnp.int32),
        pltpu.VMEM((CHUNK,), jnp.int32),
        pltpu.VMEM((CHUNK,), jnp.int32),
        pltpu.VMEM((CHUNK, D), jnp.float32),
        pltpu.VMEM((CHUNK, D), jnp.float32),
        pltpu.VMEM((CHUNK, D), jnp.float32),
        pltpu.SemaphoreType.DMA,
        pltpu.SemaphoreType.DMA,
        pltpu.SemaphoreType.DMA,
        pltpu.VMEM_SHARED((N_PAD, D), jnp.float32),
    ],
)

_sc_count = pl.kernel(
    _sc_count_body,
    out_type=jax.ShapeDtypeStruct((NUM_TILES, 1, N_PAD), jnp.float32),
    mesh=_SC_MESH,
    scratch_types=[
        pltpu.VMEM((CNT_NCH, CNT_CHUNK), jnp.int32),
        pltpu.VMEM((1, N_PAD), jnp.float32),
    ],
    compiler_params=_SC_CP,
)

_RB = 400


def _finish_body(relu, agg_ref, cnt_ref, x_ref, wl_ref, bl_ref, wr_ref, o_ref):
    agg = agg_ref[0] + agg_ref[1]
    cnt = jnp.sum(cnt_ref[...], axis=1, keepdims=True)
    mean = agg * (1.0 / jnp.maximum(cnt, 1.0))
    z = lax.dot_general(mean, wl_ref[...], (((1,), (1,)), ((), ())),
                        preferred_element_type=jnp.float32)
    z = z + bl_ref[...]
    z = z + lax.dot_general(x_ref[...], wr_ref[...], (((1,), (1,)), ((), ())),
                            preferred_element_type=jnp.float32)
    o_ref[...] = jnp.maximum(z, 0.0) if relu else z


def _make_finish(relu):
    return pl.pallas_call(
        functools.partial(_finish_body, relu),
        grid=(N // _RB,),
        in_specs=[
            pl.BlockSpec((NUM_CORES, _RB, D), lambda i: (0, i, 0)),
            pl.BlockSpec((_RB, NUM_TILES), lambda i: (i, 0)),
            pl.BlockSpec((_RB, D), lambda i: (i, 0)),
            pl.BlockSpec((D, D), lambda i: (0, 0)),
            pl.BlockSpec((1, D), lambda i: (0, 0)),
            pl.BlockSpec((D, D), lambda i: (0, 0)),
        ],
        out_specs=pl.BlockSpec((_RB, D), lambda i: (i, 0)),
        out_shape=jax.ShapeDtypeStruct((N, D), jnp.float32),
    )


_finish_relu = _make_finish(True)
_finish_plain = _make_finish(False)


def kernel(x, edge_index, Wl1, bl1, Wr1, Wl2, bl2, Wr2):
    src = edge_index[0]
    dst = edge_index[1]
    dst3d = jnp.concatenate(
        [dst, jnp.full((E_PAD - E,), N_PAD - 1, jnp.int32)]
    ).reshape(NUM_TILES, CNT_NCH, CNT_CHUNK)
    cntT = _sc_count(dst3d).reshape(NUM_TILES, N_PAD)[:, :N].T
    agg1 = _sc_agg(x, src, dst)
    h = _finish_relu(agg1, cntT, x, Wl1, bl1.reshape(1, D), Wr1)
    agg2 = _sc_agg(h, src, dst)
    return _finish_plain(agg2, cntT, h, Wl2, bl2.reshape(1, D), Wr2)


# speedup vs baseline: 1.0565x; 1.0565x over previous
import dataclasses
import functools

import jax
import jax.numpy as jnp
from jax import lax
from jax.experimental import pallas as pl
from jax.experimental.pallas import tpu as pltpu
from jax.experimental.pallas import tpu_sc as plsc

N = 10000
E = 320000
D = 128

NUM_CORES = 2
NUM_SUBCORES = 16
NUM_TILES = NUM_CORES * NUM_SUBCORES
CHUNK = 80
NCH = 125
EPT = CHUNK * NCH
CNT_CHUNK = 128
CNT_NCH = 80
E_PAD = NUM_TILES * CNT_NCH * CNT_CHUNK
N_PAD = 10240
ROWS_PER_TILE = N_PAD // NUM_SUBCORES


def _sc_agg_body(x_hbm, src_hbm, dst_hbm, agg_out,
                 isrc0, isrc1, idst0, idst1,
                 rows0, rows1, sem_g, agg_sh):
    cid = lax.axis_index("c")
    sid = lax.axis_index("s")
    wid = cid * NUM_SUBCORES + sid
    ebase = wid * EPT

    zero16 = jnp.zeros((16,), jnp.float32)

    @pl.loop(0, CHUNK)
    def _(r):
        @pl.loop(0, D // 16)
        def _(c):
            rows0[r, pl.ds(c * 16, 16)] = zero16

    @pl.loop(0, ROWS_PER_TILE // CHUNK)
    def _(k):
        pltpu.sync_copy(rows0, agg_sh.at[pl.ds(sid * ROWS_PER_TILE + k * CHUNK, CHUNK)])

    plsc.subcore_barrier()

    def load(hbm, j, buf):
        pltpu.sync_copy(hbm.at[pl.ds(ebase + j * CHUNK, CHUNK)], buf)

    def gather(idx_buf, dst_buf):
        return pltpu.async_copy(x_hbm.at[idx_buf], dst_buf, sem_g)

    def scatter(buf, idx_buf):
        pltpu.sync_copy(buf, agg_sh.at[idx_buf], add=True)

    load(src_hbm, 0, isrc0)
    load(dst_hbm, 0, idst0)
    pltpu.sync_copy(x_hbm.at[isrc0], rows0)
    load(src_hbm, 1, isrc1)
    load(dst_hbm, 1, idst1)

    @pl.loop(0, NCH // 2)
    def _(jj):
        j = jj * 2
        cp1 = gather(isrc1, rows1)
        scatter(rows0, idst0)
        load(src_hbm, jnp.minimum(j + 2, NCH - 1), isrc0)
        load(dst_hbm, jnp.minimum(j + 2, NCH - 1), idst0)
        cp1.wait()
        cp2 = gather(isrc0, rows0)
        scatter(rows1, idst1)
        load(src_hbm, jnp.minimum(j + 3, NCH - 1), isrc1)
        load(dst_hbm, jnp.minimum(j + 3, NCH - 1), idst1)
        cp2.wait()

    scatter(rows0, idst0)

    plsc.subcore_barrier()

    pltpu.sync_copy(agg_sh.at[pl.ds(sid * ROWS_PER_TILE, ROWS_PER_TILE)],
                    agg_out.at[cid, pl.ds(sid * ROWS_PER_TILE, ROWS_PER_TILE)])


def _sc_count_body(dst_hbm, cnt_out, idx_dst, cnt_local):
    cid = lax.axis_index("c")
    sid = lax.axis_index("s")
    wid = cid * NUM_SUBCORES + sid

    zero16 = jnp.zeros((16,), jnp.float32)
    zeros16i = jnp.zeros((16,), jnp.int32)
    ones16 = jnp.ones((16,), jnp.float32)

    @pl.loop(0, N_PAD // 16)
    def _(r):
        cnt_local[0, pl.ds(r * 16, 16)] = zero16

    pltpu.sync_copy(dst_hbm.at[wid], idx_dst)

    @pl.loop(0, CNT_NCH)
    def _(j):
        @pl.loop(0, CNT_CHUNK // 16)
        def _(t):
            idx16 = idx_dst[j, pl.ds(t * 16, 16)]
            plsc.addupdate_scatter(cnt_local, [zeros16i, idx16], ones16)

    pltpu.sync_copy(cnt_local, cnt_out.at[wid])


_SC_MESH = plsc.VectorSubcoreMesh(core_axis_name="c", subcore_axis_name="s")

_SC_CP = pltpu.CompilerParams()
if "needs_layout_passes" in pltpu.CompilerParams.__dataclass_fields__:
    _SC_CP = dataclasses.replace(_SC_CP, needs_layout_passes=False)

_sc_agg = pl.kernel(
    _sc_agg_body,
    out_type=jax.ShapeDtypeStruct((NUM_CORES, N_PAD, D), jnp.float32),
    mesh=_SC_MESH,
    scratch_types=[
        pltpu.VMEM((CHUNK,), jnp.int32),
        pltpu.VMEM((CHUNK,), jnp.int32),
        pltpu.VMEM((CHUNK,), jnp.int32),
        pltpu.VMEM((CHUNK,), jnp.int32),
        pltpu.VMEM((CHUNK, D), jnp.float32),
        pltpu.VMEM((CHUNK, D), jnp.float32),
        pltpu.SemaphoreType.DMA,
        pltpu.VMEM_SHARED((N_PAD, D), jnp.float32),
    ],
)

_sc_count = pl.kernel(
    _sc_count_body,
    out_type=jax.ShapeDtypeStruct((NUM_TILES, 1, N_PAD), jnp.float32),
    mesh=_SC_MESH,
    scratch_types=[
        pltpu.VMEM((CNT_NCH, CNT_CHUNK), jnp.int32),
        pltpu.VMEM((1, N_PAD), jnp.float32),
    ],
    compiler_params=_SC_CP,
)

_RB = 400


def _finish_body(relu, agg_ref, cnt_ref, x_ref, wl_ref, bl_ref, wr_ref, o_ref):
    agg = agg_ref[0] + agg_ref[1]
    cnt = jnp.sum(cnt_ref[...], axis=1, keepdims=True)
    mean = agg * (1.0 / jnp.maximum(cnt, 1.0))
    z = lax.dot_general(mean, wl_ref[...], (((1,), (1,)), ((), ())),
                        preferred_element_type=jnp.float32)
    z = z + bl_ref[...]
    z = z + lax.dot_general(x_ref[...], wr_ref[...], (((1,), (1,)), ((), ())),
                            preferred_element_type=jnp.float32)
    o_ref[...] = jnp.maximum(z, 0.0) if relu else z


def _make_finish(relu):
    return pl.pallas_call(
        functools.partial(_finish_body, relu),
        grid=(N // _RB,),
        in_specs=[
            pl.BlockSpec((NUM_CORES, _RB, D), lambda i: (0, i, 0)),
            pl.BlockSpec((_RB, NUM_TILES), lambda i: (i, 0)),
            pl.BlockSpec((_RB, D), lambda i: (i, 0)),
            pl.BlockSpec((D, D), lambda i: (0, 0)),
            pl.BlockSpec((1, D), lambda i: (0, 0)),
            pl.BlockSpec((D, D), lambda i: (0, 0)),
        ],
        out_specs=pl.BlockSpec((_RB, D), lambda i: (i, 0)),
        out_shape=jax.ShapeDtypeStruct((N, D), jnp.float32),
    )


_finish_relu = _make_finish(True)
_finish_plain = _make_finish(False)


def kernel(x, edge_index, Wl1, bl1, Wr1, Wl2, bl2, Wr2):
    src = edge_index[0]
    dst = edge_index[1]
    dst3d = jnp.concatenate(
        [dst, jnp.full((E_PAD - E,), N_PAD - 1, jnp.int32)]
    ).reshape(NUM_TILES, CNT_NCH, CNT_CHUNK)
    cntT = _sc_count(dst3d).reshape(NUM_TILES, N_PAD)[:, :N].T
    agg1 = _sc_agg(x, src, dst)
    h = _finish_relu(agg1, cntT, x, Wl1, bl1.reshape(1, D), Wr1)
    agg2 = _sc_agg(h, src, dst)
    return _finish_plain(agg2, cntT, h, Wl2, bl2.reshape(1, D), Wr2)
